# Initial kernel scaffold; baseline (speedup 1.0000x reference)
import jax
import jax.numpy as jnp
from jax.experimental import pallas as pl


def kernel(x, edge_index, W1, b1, W2, b2):
    raise NotImplementedError("write your pallas kernel here")


# speedup vs baseline: 12.3965x; 12.3965x over previous
import functools

import jax
import jax.numpy as jnp
from jax import lax
from jax.experimental import pallas as pl
from jax.experimental.pallas import tpu as pltpu
from jax.experimental.pallas import tpu_sc as plsc

NW = 32
K_EDGE = 80
LANES = 16


def _mesh():
    return plsc.VectorSubcoreMesh(core_axis_name="c", subcore_axis_name="s")


def _deg_body(ew, dst_hbm, out_hbm, deg_v, idx_v):
    c = lax.axis_index("c")
    s = lax.axis_index("s")
    wid = c * 16 + s
    nchunks = ew // K_EDGE

    npad = deg_v.shape[0]
    zero16 = jnp.zeros((LANES,), jnp.float32)
    def zstep(i, _):
        deg_v[pl.ds(i * LANES, LANES)] = zero16
        return 0
    lax.fori_loop(0, npad // LANES, zstep, 0)

    ones16 = jnp.ones((LANES,), jnp.float32)
    base_w = wid * ew

    def step(g, _):
        pltpu.sync_copy(dst_hbm.at[pl.ds(base_w + g * K_EDGE, K_EDGE)], idx_v)
        def inner(j, _):
            idx = idx_v[pl.ds(j * LANES, LANES)]
            plsc.addupdate_scatter(deg_v, [idx], ones16)
            return 0
        lax.fori_loop(0, K_EDGE // LANES, inner, 0)
        return 0

    lax.fori_loop(0, nchunks, step, 0)
    pltpu.sync_copy(deg_v, out_hbm.at[wid])


def _deg_partials(dst, npad):
    ew = dst.shape[0] // NW
    kern = pl.kernel(
        functools.partial(_deg_body, ew),
        out_type=jax.ShapeDtypeStruct((NW, npad), jnp.float32),
        mesh=_mesh(),
        scratch_types=[
            pltpu.VMEM((npad,), jnp.float32),
            pltpu.VMEM((K_EDGE,), jnp.int32),
        ],
        compiler_params=pltpu.CompilerParams(needs_layout_passes=False),
    )
    return kern(dst)


def _agg_body(ew, d, npad, src_hbm, dst_hbm, hp_hbm, out_hbm,
              acc_sh, sidx_v, didx_v, rows_v, sem):
    c = lax.axis_index("c")
    s = lax.axis_index("s")
    wid = c * 16 + s
    nchunks = ew // K_EDGE
    rows_per_sub = npad // 16

    zero16 = jnp.zeros((LANES,), jnp.float32)
    def zstep(i, _):
        r = i // (d // LANES)
        col = i % (d // LANES)
        rows_v[r, pl.ds(col * LANES, LANES)] = zero16
        return 0
    lax.fori_loop(0, K_EDGE * d // LANES, zstep, 0)

    def zacc(i, _):
        pltpu.sync_copy(rows_v, acc_sh.at[pl.ds(s * rows_per_sub + i * K_EDGE,
                                                K_EDGE)])
        return 0
    lax.fori_loop(0, rows_per_sub // K_EDGE, zacc, 0)
    plsc.subcore_barrier()

    base_w = wid * ew

    def step(g, _):
        base = base_w + g * K_EDGE
        pltpu.sync_copy(src_hbm.at[pl.ds(base, K_EDGE)], sidx_v)
        pltpu.sync_copy(dst_hbm.at[pl.ds(base, K_EDGE)], didx_v)
        pltpu.async_copy(hp_hbm.at[sidx_v], rows_v, sem).wait()
        pltpu.sync_copy(rows_v, acc_sh.at[didx_v], add=True)
        return 0

    lax.fori_loop(0, nchunks, step, 0)
    plsc.subcore_barrier()

    pltpu.sync_copy(acc_sh.at[pl.ds(s * rows_per_sub, rows_per_sub)],
                    out_hbm.at[c, pl.ds(s * rows_per_sub, rows_per_sub)])


def _aggregate(src, dst, hp, npad):
    ew = src.shape[0] // NW
    d = hp.shape[1]
    kern = pl.kernel(
        functools.partial(_agg_body, ew, d, npad),
        out_type=jax.ShapeDtypeStruct((2, npad, d), jnp.float32),
        mesh=_mesh(),
        scratch_types=[
            pltpu.VMEM_SHARED((npad, d), jnp.float32),
            pltpu.VMEM((K_EDGE,), jnp.int32),
            pltpu.VMEM((K_EDGE,), jnp.int32),
            pltpu.VMEM((K_EDGE, d), jnp.float32),
            pltpu.SemaphoreType.DMA,
        ],
        compiler_params=pltpu.CompilerParams(needs_layout_passes=False),
    )
    return kern(src, dst, hp)


def _tc_a_body(x_b, degt_b, w1_b, hp_b, dis_b):
    deg = jnp.sum(degt_b[...], axis=1, keepdims=True) + 1.0
    dis = lax.rsqrt(deg)
    h = jnp.dot(x_b[...], w1_b[...], preferred_element_type=jnp.float32)
    hp_b[...] = dis * h
    dis_b[...] = dis


def _tc_a(xp, degt, w1, npad, rblk):
    din, dhid = w1.shape
    grid = (npad // rblk,)
    return pl.pallas_call(
        _tc_a_body,
        grid=grid,
        in_specs=[
            pl.BlockSpec((rblk, din), lambda i: (i, 0)),
            pl.BlockSpec((rblk, NW), lambda i: (i, 0)),
            pl.BlockSpec((din, dhid), lambda i: (0, 0)),
        ],
        out_specs=[
            pl.BlockSpec((rblk, dhid), lambda i: (i, 0)),
            pl.BlockSpec((rblk, 1), lambda i: (i, 0)),
        ],
        out_shape=[
            jax.ShapeDtypeStruct((npad, dhid), jnp.float32),
            jax.ShapeDtypeStruct((npad, 1), jnp.float32),
        ],
    )(xp, degt, w1)


def _tc_b_body(p_b, hp1_b, dis_b, b1_b, w2_b, hp2_b):
    agg = p_b[0] + p_b[1] + hp1_b[...]
    z = jnp.maximum(dis_b[...] * agg + b1_b[...], 0.0)
    h2 = jnp.dot(z, w2_b[...], preferred_element_type=jnp.float32)
    hp2_b[...] = dis_b[...] * h2


def _tc_b(p1, hp1, dis, b1, w2, npad, rblk):
    dhid, dout = w2.shape
    grid = (npad // rblk,)
    return pl.pallas_call(
        _tc_b_body,
        grid=grid,
        in_specs=[
            pl.BlockSpec((2, rblk, dhid), lambda i: (0, i, 0)),
            pl.BlockSpec((rblk, dhid), lambda i: (i, 0)),
            pl.BlockSpec((rblk, 1), lambda i: (i, 0)),
            pl.BlockSpec((1, dhid), lambda i: (0, 0)),
            pl.BlockSpec((dhid, dout), lambda i: (0, 0)),
        ],
        out_specs=pl.BlockSpec((rblk, dout), lambda i: (i, 0)),
        out_shape=jax.ShapeDtypeStruct((npad, dout), jnp.float32),
    )(p1, hp1, dis, b1, w2)


def _tc_c_body(q_b, hp2_b, dis_b, b2_b, out_b):
    agg = q_b[0] + q_b[1] + hp2_b[...]
    out_b[...] = dis_b[...] * agg + b2_b[...]


def _tc_c(q, hp2, dis, b2, npad, rblk):
    dout = hp2.shape[1]
    grid = (npad // rblk,)
    return pl.pallas_call(
        _tc_c_body,
        grid=grid,
        in_specs=[
            pl.BlockSpec((2, rblk, dout), lambda i: (0, i, 0)),
            pl.BlockSpec((rblk, dout), lambda i: (i, 0)),
            pl.BlockSpec((rblk, 1), lambda i: (i, 0)),
            pl.BlockSpec((1, dout), lambda i: (0, 0)),
        ],
        out_specs=pl.BlockSpec((rblk, dout), lambda i: (i, 0)),
        out_shape=jax.ShapeDtypeStruct((npad, dout), jnp.float32),
    )(q, hp2, dis, b2)


def kernel(x, edge_index, W1, b1, W2, b2):
    n, din = x.shape
    e = edge_index.shape[1]
    rblk = 256
    npad = ((n + 1) + 2560 - 1) // 2560 * 2560

    src = edge_index[0].astype(jnp.int32)
    dst = edge_index[1].astype(jnp.int32)
    echunk = NW * K_EDGE
    ep = (e + echunk - 1) // echunk * echunk
    if ep != e:
        src = jnp.pad(src, (0, ep - e), constant_values=n)
        dst = jnp.pad(dst, (0, ep - e), constant_values=n)

    xp = jnp.pad(x, ((0, npad - n), (0, 0)))
    dout = W2.shape[1]
    dpad = ((dout + 127) // 128) * 128
    w2p = jnp.pad(W2, ((0, 0), (0, dpad - dout)))
    b1r = b1.reshape(1, -1)
    b2r = jnp.pad(b2, (0, dpad - dout)).reshape(1, -1)

    degp = _deg_partials(dst, npad)
    degt = degp.T
    hp1, dis = _tc_a(xp, degt, W1, npad, rblk)
    p1 = _aggregate(src, dst, hp1, npad)
    hp2 = _tc_b(p1, hp1, dis, b1r, w2p, npad, rblk)
    q = _aggregate(src, dst, hp2, npad)
    outp = _tc_c(q, hp2, dis, b2r, npad, rblk)
    return outp[:n, :dout]
